# baseline (speedup 1.0000x reference)
import jax
import jax.numpy as jnp
from jax import lax
from jax.experimental import pallas as pl
from jax.experimental.pallas import tpu as pltpu
from jax.experimental.pallas import tpu_sc as plsc

B = 16384
D = 32
PACK = 128 // D
NC = 2
NS = 16
NW = NC * NS
BPW = B // NW
CH = 128
NCH = BPW // CH
NGRP = CH // 16


def _gather_body(iidx_hbm, uidx_hbm, item_tab, user_tab, dp_hbm,
                 iidx_v, uidx_v, irow_v, urow_v,
                 ibuf0, ibuf1, ubuf0, ubuf1, dp_v, sem):
    wid = lax.axis_index("s") * NC + lax.axis_index("c")
    pltpu.sync_copy(iidx_hbm.at[pl.ds(wid * NCH, NCH)], iidx_v)
    pltpu.sync_copy(uidx_hbm.at[pl.ds(wid * NCH, NCH)], uidx_v)

    for c in range(NCH):
        for g in range(NGRP):
            s = pl.ds(g * 16, 16)
            irow_v[c, s] = lax.shift_right_logical(iidx_v[c, s], 2)
            urow_v[c, s] = lax.shift_right_logical(uidx_v[c, s], 2)

    ibufs = [ibuf0, ibuf1]
    ubufs = [ubuf0, ubuf1]

    def fire(c):
        return (
            pltpu.async_copy(item_tab.at[irow_v.at[c]], ibufs[c % 2], sem),
            pltpu.async_copy(user_tab.at[urow_v.at[c]], ubufs[c % 2], sem),
        )

    lane = jnp.arange(16, dtype=jnp.int32)
    pending = fire(0)
    for c in range(NCH):
        nxt = fire(c + 1) if c + 1 < NCH else None
        for cp in pending:
            cp.wait()
        ibuf = ibufs[c % 2]
        ubuf = ubufs[c % 2]

        def grp(g, _):
            s = pl.dslice(g * 16, 16)
            idxi = iidx_v[c, s]
            idxu = uidx_v[c, s]
            ioff = (idxi & 3) * D
            uoff = (idxu & 3) * D
            blocal = g * 16 + lane
            bvec = c * CH + blocal
            brow = lax.shift_right_logical(bvec, 2)
            bcol = (bvec & 3) * D
            for d in range(D):
                icol = plsc.load_gather(ibuf, [blocal, ioff + d])
                ucol = plsc.load_gather(ubuf, [blocal, uoff + d])
                plsc.store_scatter(dp_v, [brow, bcol + d], icol * ucol)
            return 0

        lax.fori_loop(0, NGRP, grp, 0)
        pending = nxt

    pltpu.sync_copy(dp_v, dp_hbm.at[pl.ds(wid * (BPW // PACK), BPW // PACK)])


_gather = pl.kernel(
    _gather_body,
    mesh=plsc.VectorSubcoreMesh(core_axis_name="c", subcore_axis_name="s"),
    out_type=jax.ShapeDtypeStruct((B // PACK, 128), jnp.float32),
    scratch_types=[
        pltpu.VMEM((NCH, CH), jnp.int32),
        pltpu.VMEM((NCH, CH), jnp.int32),
        pltpu.VMEM((NCH, CH), jnp.int32),
        pltpu.VMEM((NCH, CH), jnp.int32),
        pltpu.VMEM((CH, 128), jnp.float32),
        pltpu.VMEM((CH, 128), jnp.float32),
        pltpu.VMEM((CH, 128), jnp.float32),
        pltpu.VMEM((CH, 128), jnp.float32),
        pltpu.VMEM((BPW // PACK, 128), jnp.float32),
        pltpu.SemaphoreType.DMA,
    ],
    compiler_params=pltpu.CompilerParams(needs_layout_passes=False),
)


def _mlp_body(dp_ref, w4_ref, b4_ref, out_ref):
    acc = jnp.dot(dp_ref[...], w4_ref[...], preferred_element_type=jnp.float32)
    out_ref[...] = jax.nn.sigmoid(acc + b4_ref[...])


_BB = 512


_mlp = pl.pallas_call(
    _mlp_body,
    grid=(B // PACK // _BB,),
    in_specs=[
        pl.BlockSpec((_BB, 128), lambda i: (i, 0)),
        pl.BlockSpec((128, 128), lambda i: (0, 0)),
        pl.BlockSpec((1, 128), lambda i: (0, 0)),
    ],
    out_specs=pl.BlockSpec((_BB, 128), lambda i: (i, 0)),
    out_shape=jax.ShapeDtypeStruct((B // PACK, 128), jnp.float32),
)


def kernel(item_vec, user_vec, item_table, user_table, W, b):
    tab_i = item_table.reshape(-1, 128)
    tab_u = user_table.reshape(-1, 128)
    iidx = item_vec.astype(jnp.int32).reshape(NW * NCH, CH)
    uidx = user_vec.astype(jnp.int32).reshape(NW * NCH, CH)
    dp4 = _gather(iidx, uidx, tab_i, tab_u)
    eye4 = jnp.eye(PACK, dtype=jnp.float32)
    w4 = jnp.einsum("pq,kd->pkqd", eye4, W.T).reshape(128, 128)
    b4 = jnp.tile(b, PACK).reshape(1, 128)
    out4 = _mlp(dp4, w4, b4)
    return out4.reshape(B, D)


# speedup vs baseline: 3.9372x; 3.9372x over previous
import jax
import jax.numpy as jnp
from jax import lax
from jax.experimental import pallas as pl
from jax.experimental.pallas import tpu as pltpu
from jax.experimental.pallas import tpu_sc as plsc

B = 16384
D = 32
NC = 2
NS = 16
NW = NC * NS
BPW = B // NW
RING = 4


def _gather_body(iidx_hbm, uidx_hbm, tabi, tabu, dp_hbm,
                 iidx_s, uidx_s, ibuf, ubuf, dp_v, isem, usem):
    wid = lax.axis_index("s") * NC + lax.axis_index("c")
    base = wid * BPW
    pltpu.sync_copy(iidx_hbm.at[pl.ds(base, BPW)], iidx_s.at[pl.ds(0, BPW)])
    pltpu.sync_copy(uidx_hbm.at[pl.ds(base, BPW)], uidx_s.at[pl.ds(0, BPW)])

    def sidx(ref, k):
        return ref[pl.ds(k, 16)][0]

    def fire(k, r):
        ci = pl.multiple_of((sidx(iidx_s, k) >> 7) * 128, 128)
        cu = pl.multiple_of((sidx(uidx_s, k) >> 7) * 128, 128)
        pltpu.async_copy(tabi.at[:, pl.ds(ci, 128)], ibuf.at[r], isem.at[r])
        pltpu.async_copy(tabu.at[:, pl.ds(cu, 128)], ubuf.at[r], usem.at[r])

    for k in range(RING - 1):
        fire(k, k)

    rows0 = jnp.arange(16, dtype=jnp.int32)
    rows1 = rows0 + 16

    def step(n, _):
        r = lax.rem(n, RING)
        nf = n + (RING - 1)

        @pl.when(nf < BPW)
        def _():
            fire(nf, lax.rem(nf, RING))

        pltpu.make_async_copy(tabi.at[:, pl.ds(0, 128)], ibuf.at[r], isem.at[r]).wait()
        pltpu.make_async_copy(tabu.at[:, pl.ds(0, 128)], ubuf.at[r], usem.at[r]).wait()

        wi = jnp.full((16,), sidx(iidx_s, n) & 127, jnp.int32)
        wu = jnp.full((16,), sidx(uidx_s, n) & 127, jnp.int32)
        pos = jnp.full((16,), n, jnp.int32)
        i0 = plsc.load_gather(ibuf.at[r], [rows0, wi])
        i1 = plsc.load_gather(ibuf.at[r], [rows1, wi])
        u0 = plsc.load_gather(ubuf.at[r], [rows0, wu])
        u1 = plsc.load_gather(ubuf.at[r], [rows1, wu])
        plsc.store_scatter(dp_v, [rows0, pos], i0 * u0)
        plsc.store_scatter(dp_v, [rows1, pos], i1 * u1)
        return 0

    lax.fori_loop(0, BPW, step, 0)
    pltpu.sync_copy(dp_v, dp_hbm.at[:, pl.ds(base, BPW)])


_gather = pl.kernel(
    _gather_body,
    mesh=plsc.VectorSubcoreMesh(core_axis_name="c", subcore_axis_name="s"),
    out_type=jax.ShapeDtypeStruct((D, B), jnp.float32),
    scratch_types=[
        pltpu.VMEM((BPW + 16,), jnp.int32),
        pltpu.VMEM((BPW + 16,), jnp.int32),
        pltpu.VMEM((RING, D, 128), jnp.float32),
        pltpu.VMEM((RING, D, 128), jnp.float32),
        pltpu.VMEM((D, BPW), jnp.float32),
        pltpu.SemaphoreType.DMA((RING,)),
        pltpu.SemaphoreType.DMA((RING,)),
    ],
    compiler_params=pltpu.CompilerParams(
        needs_layout_passes=False, use_tc_tiling_on_sc=True),
)


def _mlp_body(dp_ref, w_ref, b_ref, out_ref):
    acc = jnp.dot(w_ref[...], dp_ref[...], preferred_element_type=jnp.float32)
    out_ref[...] = jax.nn.sigmoid(acc + b_ref[...])


_CB = 2048


_mlp = pl.pallas_call(
    _mlp_body,
    grid=(B // _CB,),
    in_specs=[
        pl.BlockSpec((D, _CB), lambda i: (0, i)),
        pl.BlockSpec((D, D), lambda i: (0, 0)),
        pl.BlockSpec((D, 1), lambda i: (0, 0)),
    ],
    out_specs=pl.BlockSpec((D, _CB), lambda i: (0, i)),
    out_shape=jax.ShapeDtypeStruct((D, B), jnp.float32),
)


def kernel(item_vec, user_vec, item_table, user_table, W, b):
    iidx = item_vec.astype(jnp.int32)
    uidx = user_vec.astype(jnp.int32)
    dp_t = _gather(iidx, uidx, item_table.T, user_table.T)
    out_t = _mlp(dp_t, W, b.reshape(D, 1))
    return out_t.T


# speedup vs baseline: 4.0363x; 1.0252x over previous
import jax
import jax.numpy as jnp
from jax import lax
from jax.experimental import pallas as pl
from jax.experimental.pallas import tpu as pltpu
from jax.experimental.pallas import tpu_sc as plsc

B = 16384
D = 32
NC = 2
NS = 16
NW = NC * NS
BPW = B // NW
RING = 8


def _gather_body(iidx_hbm, uidx_hbm, tabi, tabu, dp_hbm,
                 iidx_s, uidx_s, ibuf, ubuf, dp_v, isem, usem):
    wid = lax.axis_index("s") * NC + lax.axis_index("c")
    base = wid * BPW
    pltpu.sync_copy(iidx_hbm.at[pl.ds(base, BPW)], iidx_s.at[pl.ds(0, BPW)])
    pltpu.sync_copy(uidx_hbm.at[pl.ds(base, BPW)], uidx_s.at[pl.ds(0, BPW)])

    def sidx(ref, k):
        return ref[pl.ds(k, 16)][0]

    def fire(k, r):
        ci = pl.multiple_of((sidx(iidx_s, k) >> 7) * 128, 128)
        cu = pl.multiple_of((sidx(uidx_s, k) >> 7) * 128, 128)
        pltpu.async_copy(tabi.at[:, pl.ds(ci, 128)], ibuf.at[r], isem.at[r])
        pltpu.async_copy(tabu.at[:, pl.ds(cu, 128)], ubuf.at[r], usem.at[r])

    for k in range(RING - 1):
        fire(k, k)

    rows0 = jnp.arange(16, dtype=jnp.int32)
    rows1 = rows0 + 16

    def step(n, _):
        r = lax.rem(n, RING)
        nf = n + (RING - 1)

        @pl.when(nf < BPW)
        def _():
            fire(nf, lax.rem(nf, RING))

        pltpu.make_async_copy(tabi.at[:, pl.ds(0, 128)], ibuf.at[r], isem.at[r]).wait()
        pltpu.make_async_copy(tabu.at[:, pl.ds(0, 128)], ubuf.at[r], usem.at[r]).wait()

        wi = jnp.full((16,), sidx(iidx_s, n) & 127, jnp.int32)
        wu = jnp.full((16,), sidx(uidx_s, n) & 127, jnp.int32)
        pos = jnp.full((16,), n, jnp.int32)
        i0 = plsc.load_gather(ibuf.at[r], [rows0, wi])
        i1 = plsc.load_gather(ibuf.at[r], [rows1, wi])
        u0 = plsc.load_gather(ubuf.at[r], [rows0, wu])
        u1 = plsc.load_gather(ubuf.at[r], [rows1, wu])
        plsc.store_scatter(dp_v, [rows0, pos], i0 * u0)
        plsc.store_scatter(dp_v, [rows1, pos], i1 * u1)
        return 0

    lax.fori_loop(0, BPW, step, 0)
    pltpu.sync_copy(dp_v, dp_hbm.at[:, pl.ds(base, BPW)])


_gather = pl.kernel(
    _gather_body,
    mesh=plsc.VectorSubcoreMesh(core_axis_name="c", subcore_axis_name="s"),
    out_type=jax.ShapeDtypeStruct((D, B), jnp.float32),
    scratch_types=[
        pltpu.VMEM((BPW + 16,), jnp.int32),
        pltpu.VMEM((BPW + 16,), jnp.int32),
        pltpu.VMEM((RING, D, 128), jnp.float32),
        pltpu.VMEM((RING, D, 128), jnp.float32),
        pltpu.VMEM((D, BPW), jnp.float32),
        pltpu.SemaphoreType.DMA((RING,)),
        pltpu.SemaphoreType.DMA((RING,)),
    ],
    compiler_params=pltpu.CompilerParams(
        needs_layout_passes=False, use_tc_tiling_on_sc=True),
)


def _mlp_body(dp_ref, w_ref, b_ref, out_ref):
    acc = jnp.dot(w_ref[...], dp_ref[...], preferred_element_type=jnp.float32)
    out_ref[...] = jax.nn.sigmoid(acc + b_ref[...])


_CB = 2048


_mlp = pl.pallas_call(
    _mlp_body,
    grid=(B // _CB,),
    in_specs=[
        pl.BlockSpec((D, _CB), lambda i: (0, i)),
        pl.BlockSpec((D, D), lambda i: (0, 0)),
        pl.BlockSpec((D, 1), lambda i: (0, 0)),
    ],
    out_specs=pl.BlockSpec((D, _CB), lambda i: (0, i)),
    out_shape=jax.ShapeDtypeStruct((D, B), jnp.float32),
)


def kernel(item_vec, user_vec, item_table, user_table, W, b):
    iidx = item_vec.astype(jnp.int32)
    uidx = user_vec.astype(jnp.int32)
    dp_t = _gather(iidx, uidx, item_table.T, user_table.T)
    out_t = _mlp(dp_t, W, b.reshape(D, 1))
    return out_t.T
